# baseline (speedup 1.0000x reference)
import functools

import jax
import jax.numpy as jnp
from jax import lax
from jax.experimental import pallas as pl
from jax.experimental.pallas import tpu as pltpu
from jax.experimental.pallas import tpu_sc as plsc

N = 2048
H = 8
DH = 64
F = 512
NIT = 8
TOPK = 10
BLK = 256
NNCLS = 64


def _proj_body(fea_ref, wq_ref, bq_ref, wk_ref, bk_ref, qs_ref, k_ref):
    fea = fea_ref[...]
    dn = (((1,), (1,)), ((), ()))
    q = jax.lax.dot_general(fea, wq_ref[...], dn,
                            preferred_element_type=jnp.float32) + bq_ref[...]
    k = jax.lax.dot_general(fea, wk_ref[...], dn,
                            preferred_element_type=jnp.float32) + bk_ref[...]
    s = jnp.sqrt(jnp.sum(q * q) * jnp.sum(k * k))
    ks_sum = jnp.sum(k, axis=0, keepdims=True)
    scales = []
    for h in range(H):
        sl = slice(h * DH, (h + 1) * DH)
        dh = jnp.sum(q[:, sl] * ks_sum[:, sl], axis=1, keepdims=True)
        c = 1.0 / (H * (dh + N * s))
        scales.append(jnp.broadcast_to(c, (N, DH)))
    qs_ref[...] = q * jnp.concatenate(scales, axis=1)
    k_ref[...] = k


def _attn_body(qs_ref, kf_ref, gum_ref, gs_ref):
    logits = jax.lax.dot_general(qs_ref[...], kf_ref[...],
                                 (((1,), (1,)), ((), ())),
                                 preferred_element_type=jnp.float32)
    logits = logits + gum_ref[...]
    m = jnp.max(logits, axis=1, keepdims=True)
    e = jnp.exp(logits - m)
    gs_ref[...] = e / jnp.sum(e, axis=1, keepdims=True)


SC_BATCH = 16
SC_NW = 32
SC_RPW = N // SC_NW
F32_MIN = float(jnp.finfo(jnp.float32).min)


def _sc_topk_body(gs_hbm, attn_hbm, buf, sem):
    c = lax.axis_index("c")
    s = lax.axis_index("s")
    wid = s * 2 + c
    row_off = lax.iota(jnp.int32, 16) * N
    lowest = jnp.full((16,), F32_MIN, jnp.float32)
    zero16 = jnp.zeros((16,), jnp.float32)

    for b in range(SC_RPW // SC_BATCH):
        row0 = wid * SC_RPW + b * SC_BATCH
        pltpu.sync_copy(gs_hbm.at[pl.ds(row0 * N, SC_BATCH * N)], buf)

        def insert(args):
            x, col, t, idx = args
            colv = jnp.full((16,), col, jnp.int32)
            hit = x > t[0]
            tl = [jnp.where(hit, x, t[0])] + list(t[1:])
            il = [jnp.where(hit, colv, idx[0])] + list(idx[1:])
            for j in range(TOPK - 1):
                gt = tl[j] > tl[j + 1]
                hi = jnp.where(gt, tl[j], tl[j + 1])
                lo = jnp.where(gt, tl[j + 1], tl[j])
                ihi = jnp.where(gt, il[j], il[j + 1])
                ilo = jnp.where(gt, il[j + 1], il[j])
                tl[j], tl[j + 1] = lo, hi
                il[j], il[j + 1] = ilo, ihi
            return tuple(tl), tuple(il)

        def col_step(col, carry):
            t, idx = carry
            x = plsc.load_gather(buf, [row_off + col])
            hit = jnp.max((x > t[0]).astype(jnp.int32)) > 0
            return lax.cond(hit, insert,
                            lambda args: (args[2], args[3]),
                            (x, col, t, idx))

        t0 = (lowest,) * TOPK
        i0 = (jnp.zeros((16,), jnp.int32),) * TOPK
        t, idx = lax.fori_loop(0, N, col_step, (t0, i0))

        def zero_step(k, carry):
            buf[pl.ds(k * 16, 16)] = zero16
            return carry

        lax.fori_loop(0, SC_BATCH * N // 16, zero_step, 0)
        for j in range(TOPK):
            plsc.store_scatter(buf, [row_off + idx[j]], t[j])

        pltpu.sync_copy(buf, attn_hbm.at[pl.ds(row0 * N, SC_BATCH * N)])


_sc_topk = pl.kernel(
    _sc_topk_body,
    out_type=jax.ShapeDtypeStruct((N * N,), jnp.float32),
    mesh=plsc.VectorSubcoreMesh(core_axis_name="c", subcore_axis_name="s"),
    scratch_types=[pltpu.VMEM((SC_BATCH * N,), jnp.float32),
                   pltpu.SemaphoreType.DMA],
    compiler_params=pltpu.CompilerParams(use_tc_tiling_on_sc=False,
                                         needs_layout_passes=False),
)


def _horner_body(a_ref, ah_ref, preds_ref, b2_ref, b1_ref, out_ref):
    dn = (((1,), (0,)), ((), ()))
    a = a_ref[...]
    tmp = preds_ref[...]
    acc = tmp * b2_ref[0, 0]
    for i in range(1, NIT):
        tmp = jax.lax.dot_general(a, tmp, dn, preferred_element_type=jnp.float32)
        tmp = jax.lax.dot_general(a, tmp, dn, preferred_element_type=jnp.float32)
        acc = acc + tmp * b2_ref[0, i]
    ah = ah_ref[...]
    tmp = acc
    acc = tmp * b1_ref[0, 0]
    for i in range(1, NIT):
        tmp = jax.lax.dot_general(ah, tmp, dn, preferred_element_type=jnp.float32)
        acc = acc + tmp * b1_ref[0, i]
    out_ref[...] = acc


def kernel(local_preds, idx, origin_fea, A_hat, Wq_w, Wq_b, Wk_w, Wk_b,
           lin1_w, lin2_w, gumbel):
    f32 = jnp.float32
    bq = Wq_b.reshape(1, F)
    bk = Wk_b.reshape(1, F)

    qs, k = pl.pallas_call(
        _proj_body,
        out_shape=[jax.ShapeDtypeStruct((N, F), f32),
                   jax.ShapeDtypeStruct((N, F), f32)],
    )(origin_fea, Wq_w, bq, Wk_w, bk)

    nblk = N // BLK
    gs = pl.pallas_call(
        _attn_body,
        grid=(nblk,),
        in_specs=[pl.BlockSpec((BLK, F), lambda i: (i, 0)),
                  pl.BlockSpec((N, F), lambda i: (0, 0)),
                  pl.BlockSpec((BLK, N), lambda i: (i, 0))],
        out_specs=pl.BlockSpec((BLK, N), lambda i: (i, 0)),
        out_shape=jax.ShapeDtypeStruct((N, N), f32),
    )(qs, k, gumbel)

    attn = _sc_topk(gs.reshape(N * N)).reshape(N, N)

    out = pl.pallas_call(
        _horner_body,
        in_specs=[pl.BlockSpec(memory_space=pltpu.MemorySpace.VMEM),
                  pl.BlockSpec(memory_space=pltpu.MemorySpace.VMEM),
                  pl.BlockSpec(memory_space=pltpu.MemorySpace.VMEM),
                  pl.BlockSpec(memory_space=pltpu.MemorySpace.SMEM),
                  pl.BlockSpec(memory_space=pltpu.MemorySpace.SMEM)],
        out_shape=jax.ShapeDtypeStruct((N, NNCLS), f32),
    )(attn, A_hat, local_preds, lin2_w, lin1_w)
    return out


# speedup vs baseline: 1.1230x; 1.1230x over previous
import functools

import jax
import jax.numpy as jnp
from jax import lax
from jax.experimental import pallas as pl
from jax.experimental.pallas import tpu as pltpu
from jax.experimental.pallas import tpu_sc as plsc

N = 2048
H = 8
DH = 64
F = 512
NIT = 8
TOPK = 10
BLK = 256
NNCLS = 64


def _proj_body(fea_ref, wq_ref, bq_ref, wk_ref, bk_ref, qs_ref, k_ref):
    fea = fea_ref[...]
    dn = (((1,), (1,)), ((), ()))
    q = jax.lax.dot_general(fea, wq_ref[...], dn,
                            preferred_element_type=jnp.float32) + bq_ref[...]
    k = jax.lax.dot_general(fea, wk_ref[...], dn,
                            preferred_element_type=jnp.float32) + bk_ref[...]
    s = jnp.sqrt(jnp.sum(q * q) * jnp.sum(k * k))
    ks_sum = jnp.sum(k, axis=0, keepdims=True)
    scales = []
    for h in range(H):
        sl = slice(h * DH, (h + 1) * DH)
        dh = jnp.sum(q[:, sl] * ks_sum[:, sl], axis=1, keepdims=True)
        c = 1.0 / (H * (dh + N * s))
        scales.append(jnp.broadcast_to(c, (N, DH)))
    qs_ref[...] = q * jnp.concatenate(scales, axis=1)
    k_ref[...] = k


def _attn_body(qs_ref, kf_ref, gum_ref, gs_ref):
    logits = jax.lax.dot_general(qs_ref[...], kf_ref[...],
                                 (((1,), (1,)), ((), ())),
                                 preferred_element_type=jnp.float32)
    logits = logits + gum_ref[...]
    m = jnp.max(logits, axis=1, keepdims=True)
    e = jnp.exp(logits - m)
    gs_ref[...] = e / jnp.sum(e, axis=1, keepdims=True)


SC_BATCH = 16
SC_NW = 32
SC_RPW = N // SC_NW
F32_MIN = float(jnp.finfo(jnp.float32).min)


UNROLL = 8


def _sc_topk_body(gs_hbm, attn_hbm, buf, sem):
    c = lax.axis_index("c")
    s = lax.axis_index("s")
    wid = s * 2 + c
    lowest = jnp.full((16,), F32_MIN, jnp.float32)
    lane = lax.iota(jnp.int32, 16)

    def do_row(r, carry):
        base = r * N

        def chunk_step(g, t):
            for u in range(UNROLL):
                x = buf[pl.ds(base + (g * UNROLL + u) * 16, 16)]

                def insert(args):
                    x, t = args
                    tl = [jnp.maximum(x, t[0])] + list(t[1:])
                    for j in range(TOPK - 1):
                        hi = jnp.maximum(tl[j], tl[j + 1])
                        lo = jnp.minimum(tl[j], tl[j + 1])
                        tl[j], tl[j + 1] = lo, hi
                    return tuple(tl)

                hit = jnp.max((x > t[0]).astype(jnp.int32)) > 0
                t = lax.cond(hit, insert, lambda args: args[1], (x, t))
            return t

        t = lax.fori_loop(0, N // 16 // UNROLL, chunk_step, (lowest,) * TOPK)

        tl = list(t)
        theta = None
        for _ in range(TOPK):
            m = jnp.max(tl[TOPK - 1])
            win1 = jnp.min(jnp.where(tl[TOPK - 1] == m, lane, 16))
            sel = lane == win1
            for j in range(TOPK - 1, 0, -1):
                tl[j] = jnp.where(sel, tl[j - 1], tl[j])
            tl[0] = jnp.where(sel, lowest, tl[0])
            theta = m

        def mask_step(g, carry):
            for u in range(UNROLL):
                off = base + (g * UNROLL + u) * 16
                x = buf[pl.ds(off, 16)]
                buf[pl.ds(off, 16)] = jnp.where(x >= theta, x, 0.0)
            return carry

        return lax.fori_loop(0, N // 16 // UNROLL, mask_step, carry)

    for b in range(SC_RPW // SC_BATCH):
        row0 = wid * SC_RPW + b * SC_BATCH
        pltpu.sync_copy(gs_hbm.at[pl.ds(row0 * N, SC_BATCH * N)], buf)
        lax.fori_loop(0, SC_BATCH, do_row, 0)
        pltpu.sync_copy(buf, attn_hbm.at[pl.ds(row0 * N, SC_BATCH * N)])


_sc_topk = pl.kernel(
    _sc_topk_body,
    out_type=jax.ShapeDtypeStruct((N * N,), jnp.float32),
    mesh=plsc.VectorSubcoreMesh(core_axis_name="c", subcore_axis_name="s"),
    scratch_types=[pltpu.VMEM((SC_BATCH * N,), jnp.float32),
                   pltpu.SemaphoreType.DMA],
    compiler_params=pltpu.CompilerParams(use_tc_tiling_on_sc=False,
                                         needs_layout_passes=False),
)


def _horner_body(a_ref, ah_ref, preds_ref, b2_ref, b1_ref, out_ref):
    dn = (((1,), (0,)), ((), ()))
    a = a_ref[...]
    tmp = preds_ref[...]
    acc = tmp * b2_ref[0, 0]
    for i in range(1, NIT):
        tmp = jax.lax.dot_general(a, tmp, dn, preferred_element_type=jnp.float32)
        tmp = jax.lax.dot_general(a, tmp, dn, preferred_element_type=jnp.float32)
        acc = acc + tmp * b2_ref[0, i]
    ah = ah_ref[...]
    tmp = acc
    acc = tmp * b1_ref[0, 0]
    for i in range(1, NIT):
        tmp = jax.lax.dot_general(ah, tmp, dn, preferred_element_type=jnp.float32)
        acc = acc + tmp * b1_ref[0, i]
    out_ref[...] = acc


def kernel(local_preds, idx, origin_fea, A_hat, Wq_w, Wq_b, Wk_w, Wk_b,
           lin1_w, lin2_w, gumbel):
    f32 = jnp.float32
    bq = Wq_b.reshape(1, F)
    bk = Wk_b.reshape(1, F)

    qs, k = pl.pallas_call(
        _proj_body,
        out_shape=[jax.ShapeDtypeStruct((N, F), f32),
                   jax.ShapeDtypeStruct((N, F), f32)],
    )(origin_fea, Wq_w, bq, Wk_w, bk)

    nblk = N // BLK
    gs = pl.pallas_call(
        _attn_body,
        grid=(nblk,),
        in_specs=[pl.BlockSpec((BLK, F), lambda i: (i, 0)),
                  pl.BlockSpec((N, F), lambda i: (0, 0)),
                  pl.BlockSpec((BLK, N), lambda i: (i, 0))],
        out_specs=pl.BlockSpec((BLK, N), lambda i: (i, 0)),
        out_shape=jax.ShapeDtypeStruct((N, N), f32),
    )(qs, k, gumbel)

    attn = _sc_topk(gs.reshape(N * N)).reshape(N, N)

    out = pl.pallas_call(
        _horner_body,
        in_specs=[pl.BlockSpec(memory_space=pltpu.MemorySpace.VMEM),
                  pl.BlockSpec(memory_space=pltpu.MemorySpace.VMEM),
                  pl.BlockSpec(memory_space=pltpu.MemorySpace.VMEM),
                  pl.BlockSpec(memory_space=pltpu.MemorySpace.SMEM),
                  pl.BlockSpec(memory_space=pltpu.MemorySpace.SMEM)],
        out_shape=jax.ShapeDtypeStruct((N, NNCLS), f32),
    )(attn, A_hat, local_preds, lin2_w, lin1_w)
    return out


# speedup vs baseline: 1.5758x; 1.4032x over previous
import functools

import jax
import jax.numpy as jnp
from jax import lax
from jax.experimental import pallas as pl
from jax.experimental.pallas import tpu as pltpu
from jax.experimental.pallas import tpu_sc as plsc

N = 2048
H = 8
DH = 64
F = 512
NIT = 8
TOPK = 10
BLK = 256
NNCLS = 64


def _proj_body(fea_ref, wq_ref, bq_ref, wk_ref, bk_ref, qs_ref, k_ref):
    fea = fea_ref[...]
    dn = (((1,), (1,)), ((), ()))
    q = jax.lax.dot_general(fea, wq_ref[...], dn,
                            preferred_element_type=jnp.float32) + bq_ref[...]
    k = jax.lax.dot_general(fea, wk_ref[...], dn,
                            preferred_element_type=jnp.float32) + bk_ref[...]
    s = jnp.sqrt(jnp.sum(q * q) * jnp.sum(k * k))
    ks_sum = jnp.sum(k, axis=0, keepdims=True)
    scales = []
    for h in range(H):
        sl = slice(h * DH, (h + 1) * DH)
        dh = jnp.sum(q[:, sl] * ks_sum[:, sl], axis=1, keepdims=True)
        c = 1.0 / (H * (dh + N * s))
        scales.append(jnp.broadcast_to(c, (N, DH)))
    qs_ref[...] = q * jnp.concatenate(scales, axis=1)
    k_ref[...] = k


def _attn_body(qs_ref, kf_ref, gum_ref, gs_ref):
    logits = jax.lax.dot_general(qs_ref[...], kf_ref[...],
                                 (((1,), (1,)), ((), ())),
                                 preferred_element_type=jnp.float32)
    logits = logits + gum_ref[...]
    m = jnp.max(logits, axis=1, keepdims=True)
    e = jnp.exp(logits - m)
    gs_ref[...] = e / jnp.sum(e, axis=1, keepdims=True)


SC_BATCH = 16
SC_NW = 32
SC_RPW = N // SC_NW
F32_MIN = float(jnp.finfo(jnp.float32).min)


UNROLL = 8


def _sc_topk_body(gs_hbm, attn_hbm, buf, sem):
    c = lax.axis_index("c")
    s = lax.axis_index("s")
    wid = s * 2 + c
    lowest = jnp.full((16,), F32_MIN, jnp.float32)
    lane = lax.iota(jnp.int32, 16)

    def do_row(r, carry):
        base = r * N

        def chunk_step(g, t):
            for u in range(UNROLL):
                x = buf[pl.ds(base + (g * UNROLL + u) * 16, 16)]
                tl = [jnp.maximum(x, t[0])] + list(t[1:])
                for j in range(TOPK - 1):
                    hi = jnp.maximum(tl[j], tl[j + 1])
                    lo = jnp.minimum(tl[j], tl[j + 1])
                    tl[j], tl[j + 1] = lo, hi
                t = tuple(tl)
            return t

        t = lax.fori_loop(0, N // 16 // UNROLL, chunk_step, (lowest,) * TOPK)

        tl = list(t)
        theta = None
        for _ in range(TOPK):
            m = jnp.max(tl[TOPK - 1])
            win1 = jnp.min(jnp.where(tl[TOPK - 1] == m, lane, 16))
            sel = lane == win1
            for j in range(TOPK - 1, 0, -1):
                tl[j] = jnp.where(sel, tl[j - 1], tl[j])
            tl[0] = jnp.where(sel, lowest, tl[0])
            theta = m

        def mask_step(g, carry):
            for u in range(UNROLL):
                off = base + (g * UNROLL + u) * 16
                x = buf[pl.ds(off, 16)]
                buf[pl.ds(off, 16)] = jnp.where(x >= theta, x, 0.0)
            return carry

        return lax.fori_loop(0, N // 16 // UNROLL, mask_step, carry)

    for b in range(SC_RPW // SC_BATCH):
        row0 = wid * SC_RPW + b * SC_BATCH
        pltpu.sync_copy(gs_hbm.at[pl.ds(row0 * N, SC_BATCH * N)], buf)
        lax.fori_loop(0, SC_BATCH, do_row, 0)
        pltpu.sync_copy(buf, attn_hbm.at[pl.ds(row0 * N, SC_BATCH * N)])


_sc_topk = pl.kernel(
    _sc_topk_body,
    out_type=jax.ShapeDtypeStruct((N * N,), jnp.float32),
    mesh=plsc.VectorSubcoreMesh(core_axis_name="c", subcore_axis_name="s"),
    scratch_types=[pltpu.VMEM((SC_BATCH * N,), jnp.float32),
                   pltpu.SemaphoreType.DMA],
    compiler_params=pltpu.CompilerParams(use_tc_tiling_on_sc=False,
                                         needs_layout_passes=False),
)


def _horner_body(a_ref, ah_ref, preds_ref, b2_ref, b1_ref, out_ref):
    dn = (((1,), (0,)), ((), ()))
    a = a_ref[...]
    tmp = preds_ref[...]
    acc = tmp * b2_ref[0, 0]
    for i in range(1, NIT):
        tmp = jax.lax.dot_general(a, tmp, dn, preferred_element_type=jnp.float32)
        tmp = jax.lax.dot_general(a, tmp, dn, preferred_element_type=jnp.float32)
        acc = acc + tmp * b2_ref[0, i]
    ah = ah_ref[...]
    tmp = acc
    acc = tmp * b1_ref[0, 0]
    for i in range(1, NIT):
        tmp = jax.lax.dot_general(ah, tmp, dn, preferred_element_type=jnp.float32)
        acc = acc + tmp * b1_ref[0, i]
    out_ref[...] = acc


def kernel(local_preds, idx, origin_fea, A_hat, Wq_w, Wq_b, Wk_w, Wk_b,
           lin1_w, lin2_w, gumbel):
    f32 = jnp.float32
    bq = Wq_b.reshape(1, F)
    bk = Wk_b.reshape(1, F)

    qs, k = pl.pallas_call(
        _proj_body,
        out_shape=[jax.ShapeDtypeStruct((N, F), f32),
                   jax.ShapeDtypeStruct((N, F), f32)],
    )(origin_fea, Wq_w, bq, Wk_w, bk)

    nblk = N // BLK
    gs = pl.pallas_call(
        _attn_body,
        grid=(nblk,),
        in_specs=[pl.BlockSpec((BLK, F), lambda i: (i, 0)),
                  pl.BlockSpec((N, F), lambda i: (0, 0)),
                  pl.BlockSpec((BLK, N), lambda i: (i, 0))],
        out_specs=pl.BlockSpec((BLK, N), lambda i: (i, 0)),
        out_shape=jax.ShapeDtypeStruct((N, N), f32),
    )(qs, k, gumbel)

    attn = _sc_topk(gs.reshape(N * N)).reshape(N, N)

    out = pl.pallas_call(
        _horner_body,
        in_specs=[pl.BlockSpec(memory_space=pltpu.MemorySpace.VMEM),
                  pl.BlockSpec(memory_space=pltpu.MemorySpace.VMEM),
                  pl.BlockSpec(memory_space=pltpu.MemorySpace.VMEM),
                  pl.BlockSpec(memory_space=pltpu.MemorySpace.SMEM),
                  pl.BlockSpec(memory_space=pltpu.MemorySpace.SMEM)],
        out_shape=jax.ShapeDtypeStruct((N, NNCLS), f32),
    )(attn, A_hat, local_preds, lin2_w, lin1_w)
    return out
